# Initial kernel scaffold; baseline (speedup 1.0000x reference)
import functools

import jax
import jax.numpy as jnp
from jax import lax
from jax.experimental import pallas as pl
from jax.experimental.pallas import tpu as pltpu
from jax.experimental.pallas import tpu_sc as plsc

_NC = 2
_NS = 16
_L = 16
_NW = _NC * _NS


def _stage1_body(tbl, out, buf_a, buf_b, pres, sem_a, sem_b, *, rpw, chunk):
    cid = lax.axis_index("c")
    sid = lax.axis_index("s")
    wid = sid * _NC + cid
    base = wid * rpw
    nchunk = rpw // chunk

    pres[...] = jnp.zeros((_L,), jnp.float32)
    ones = jnp.ones((_L,), jnp.float32)
    iota = lax.iota(jnp.int32, _L)
    zero = jnp.zeros((_L,), jnp.int32)

    bufs = (buf_a, buf_b)
    sems = (sem_a, sem_b)

    def copy_in(ch, buf, sem):
        src = tbl.at[pl.ds(base + ch * chunk, chunk), 2, :]
        return pltpu.make_async_copy(src, buf, sem)

    copy_in(0, bufs[0], sems[0]).start()
    for ch in range(nchunk):
        buf = bufs[ch % 2]
        copy_in(ch, buf, sems[ch % 2]).wait()
        if ch + 1 < nchunk:
            copy_in(ch + 1, bufs[(ch + 1) % 2], sems[(ch + 1) % 2]).start()

        def blk(k, carry):
            ridx = k * _L + iota
            mi_x = plsc.load_gather(buf, [ridx, zero])
            su_x = plsc.load_gather(buf, [ridx, zero + 1])
            mi_y = plsc.load_gather(buf, [ridx, zero + 2])
            su_y = plsc.load_gather(buf, [ridx, zero + 3])
            dx = jnp.abs(su_x - mi_x)
            dy = jnp.abs(su_y - mi_y)
            gx = su_x > mi_x
            gy = su_y > mi_y
            act = jnp.where(dx < 22, jnp.where(gx, 4, 3), 1)
            act = jnp.where(dx > 24, jnp.where(gx, 3, 4), act)
            act = jnp.where(dy > 2, jnp.where(gy, 5, 2), act)
            plsc.store_scatter(pres, [act], ones)
            return carry

        lax.fori_loop(0, chunk // _L, blk, 0)

    pltpu.sync_copy(pres, out.at[wid])


def _stage2_body(m_ref, o_ref):
    o_ref[...] = jnp.max(m_ref[...], axis=0, keepdims=True)[:, :6]


@jax.jit
def kernel(ram):
    n = ram.shape[0]
    rpw = n // _NW
    chunk = min(rpw, 2048)
    tbl = ram.reshape(n, 8, 16)

    mesh = plsc.VectorSubcoreMesh(core_axis_name="c", subcore_axis_name="s")
    stage1 = pl.kernel(
        functools.partial(_stage1_body, rpw=rpw, chunk=chunk),
        out_type=jax.ShapeDtypeStruct((_NW, _L), jnp.float32),
        mesh=mesh,
        scratch_types=[
            pltpu.VMEM((chunk, _L), jnp.int32),
            pltpu.VMEM((chunk, _L), jnp.int32),
            pltpu.VMEM((_L,), jnp.float32),
            pltpu.SemaphoreType.DMA,
            pltpu.SemaphoreType.DMA,
        ],
    )
    masks = stage1(tbl)

    return pl.pallas_call(
        _stage2_body,
        out_shape=jax.ShapeDtypeStruct((1, 6), jnp.float32),
    )(masks)


# speedup vs baseline: 1.7566x; 1.7566x over previous
import functools

import jax
import jax.numpy as jnp
from jax import lax
from jax.experimental import pallas as pl
from jax.experimental.pallas import tpu as pltpu
from jax.experimental.pallas import tpu_sc as plsc

_NC = 2
_NS = 16
_L = 16
_NW = _NC * _NS


def _stage1_body(tbl, out, buf_a, buf_b, pres, sem_a, sem_b, *, rpw, chunk):
    cid = lax.axis_index("c")
    sid = lax.axis_index("s")
    wid = sid * _NC + cid
    base = wid * rpw
    nchunk = rpw // chunk

    pres[...] = jnp.zeros((_L,), jnp.float32)
    ones = jnp.ones((_L,), jnp.float32)
    iota = lax.iota(jnp.int32, _L)
    zero = jnp.zeros((_L,), jnp.int32)

    bufs = (buf_a, buf_b)
    sems = (sem_a, sem_b)

    def copy_in(ch, buf, sem):
        src = tbl.at[pl.ds(base + ch * chunk, chunk), 2, :]
        return pltpu.make_async_copy(src, buf, sem)

    copy_in(0, bufs[0], sems[0]).start()
    for ch in range(nchunk):
        buf = bufs[ch % 2]
        copy_in(ch, buf, sems[ch % 2]).wait()
        if ch + 1 < nchunk:
            copy_in(ch + 1, bufs[(ch + 1) % 2], sems[(ch + 1) % 2]).start()

        def blk(k, carry):
            ridx = k * _L + iota
            mi_x = plsc.load_gather(buf, [ridx, zero])
            su_x = plsc.load_gather(buf, [ridx, zero + 1])
            mi_y = plsc.load_gather(buf, [ridx, zero + 2])
            su_y = plsc.load_gather(buf, [ridx, zero + 3])
            dx = jnp.abs(su_x - mi_x)
            dy = jnp.abs(su_y - mi_y)
            gx = su_x > mi_x
            gy = su_y > mi_y
            act = jnp.where(dx < 22, jnp.where(gx, 4, 3), 1)
            act = jnp.where(dx > 24, jnp.where(gx, 3, 4), act)
            act = jnp.where(dy > 2, jnp.where(gy, 5, 2), act)
            plsc.store_scatter(pres, [act], ones)
            return carry

        lax.fori_loop(0, chunk // _L, blk, 0)

    pltpu.sync_copy(pres, out.at[wid])


def _stage2_body(m_ref, o_ref):
    o_ref[...] = jnp.max(m_ref[...], axis=0, keepdims=True)[:, :6]


@jax.jit
def kernel(ram):
    n = ram.shape[0]
    rpw = n // _NW
    chunk = min(rpw, 2048)
    tbl = ram.reshape(n, 8, 16)

    mesh = plsc.VectorSubcoreMesh(core_axis_name="c", subcore_axis_name="s")
    stage1 = pl.kernel(
        functools.partial(_stage1_body, rpw=rpw, chunk=chunk),
        out_type=jax.ShapeDtypeStruct((_NW, _L), jnp.float32),
        mesh=mesh,
        scratch_types=[
            pltpu.VMEM((chunk, _L), jnp.int32),
            pltpu.VMEM((chunk, _L), jnp.int32),
            pltpu.VMEM((_L,), jnp.float32),
            pltpu.SemaphoreType.DMA,
            pltpu.SemaphoreType.DMA,
        ],
        compiler_params=pltpu.CompilerParams(
            needs_layout_passes=False,
            use_tc_tiling_on_sc=False,
        ),
    )
    masks = stage1(tbl)

    return pl.pallas_call(
        _stage2_body,
        out_shape=jax.ShapeDtypeStruct((1, 6), jnp.float32),
    )(masks)


# speedup vs baseline: 43.0354x; 24.4994x over previous
import functools

import jax
import jax.numpy as jnp
from jax import lax
from jax.experimental import pallas as pl
from jax.experimental.pallas import tpu as pltpu
from jax.experimental.pallas import tpu_sc as plsc

_NC = 2
_NS = 16
_L = 16
_NW = _NC * _NS


def _stage1_body(tbl, out, buf_a, buf_b, pres, sem_a, sem_b, *, rpw, chunk):
    cid = lax.axis_index("c")
    sid = lax.axis_index("s")
    wid = sid * _NC + cid
    base = wid * rpw
    nchunk = rpw // chunk

    pres[...] = jnp.zeros((_L,), jnp.float32)
    ones = jnp.ones((_L,), jnp.float32)
    iota = lax.iota(jnp.int32, _L)
    zero = jnp.zeros((_L,), jnp.int32)

    bufs = (buf_a, buf_b)
    sems = (sem_a, sem_b)

    def copy_in(ch, buf, sem):
        src = tbl.at[pl.ds(base + ch * chunk, chunk), pl.ds(32, 16)]
        return pltpu.make_async_copy(src, buf, sem)

    copy_in(0, bufs[0], sems[0]).start()
    for ch in range(nchunk):
        buf = bufs[ch % 2]
        copy_in(ch, buf, sems[ch % 2]).wait()
        if ch + 1 < nchunk:
            copy_in(ch + 1, bufs[(ch + 1) % 2], sems[(ch + 1) % 2]).start()

        def blk(k, carry):
            ridx = k * _L + iota
            mi_x = plsc.load_gather(buf, [ridx, zero])
            su_x = plsc.load_gather(buf, [ridx, zero + 1])
            mi_y = plsc.load_gather(buf, [ridx, zero + 2])
            su_y = plsc.load_gather(buf, [ridx, zero + 3])
            dx = jnp.abs(su_x - mi_x)
            dy = jnp.abs(su_y - mi_y)
            gx = su_x > mi_x
            gy = su_y > mi_y
            act = jnp.where(dx < 22, jnp.where(gx, 4, 3), 1)
            act = jnp.where(dx > 24, jnp.where(gx, 3, 4), act)
            act = jnp.where(dy > 2, jnp.where(gy, 5, 2), act)
            plsc.store_scatter(pres, [act], ones)
            return carry

        lax.fori_loop(0, chunk // _L, blk, 0)

    pltpu.sync_copy(pres, out.at[wid])


def _stage2_body(m_ref, o_ref):
    o_ref[...] = jnp.max(m_ref[...], axis=0, keepdims=True)[:, :6]


@jax.jit
def kernel(ram):
    n = ram.shape[0]
    rpw = n // _NW
    chunk = min(rpw, 2048)

    mesh = plsc.VectorSubcoreMesh(core_axis_name="c", subcore_axis_name="s")
    stage1 = pl.kernel(
        functools.partial(_stage1_body, rpw=rpw, chunk=chunk),
        out_type=jax.ShapeDtypeStruct((_NW, _L), jnp.float32),
        mesh=mesh,
        scratch_types=[
            pltpu.VMEM((chunk, _L), jnp.int32),
            pltpu.VMEM((chunk, _L), jnp.int32),
            pltpu.VMEM((_L,), jnp.float32),
            pltpu.SemaphoreType.DMA,
            pltpu.SemaphoreType.DMA,
        ],
        compiler_params=pltpu.CompilerParams(
            needs_layout_passes=False,
            use_tc_tiling_on_sc=False,
        ),
    )
    masks = stage1(ram)

    return pl.pallas_call(
        _stage2_body,
        out_shape=jax.ShapeDtypeStruct((1, 6), jnp.float32),
    )(masks)
